# baseline (speedup 1.0000x reference)
import functools

import jax
import jax.numpy as jnp
from jax import lax
from jax.experimental import pallas as pl
from jax.experimental.pallas import tpu as pltpu
from jax.experimental.pallas import tpu_sc as plsc

N_C, N_G, N_P = 20000, 20000, 10000
N = N_C + N_G + N_P
E = 800000
D_IN, D_MID, D_HID = 256, 256, 64
N_REL = 4
DH = D_HID // 2

NUM_CORES = 2
NUM_TILES = 16
LANES = 16

CHUNK = 128
NP = 51200
ROWS_PER_TILE = NP // NUM_TILES
E_PAD = 802816
EPT = E_PAD // NUM_TILES
NCHUNK = EPT // CHUNK
KW = 2
NW = NCHUNK // KW
NCB = E_PAD // CHUNK
ZROWS = 64


def _enc_body(f_ref, w1_ref, b1_ref, w2_ref, b2_ref, o_ref):
    h = jnp.maximum(
        jnp.dot(f_ref[...], w1_ref[...],
                preferred_element_type=jnp.float32) + b1_ref[...], 0.0)
    o_ref[...] = jnp.tanh(
        jnp.dot(h, w2_ref[...],
                preferred_element_type=jnp.float32) + b2_ref[...])


def _encode_type(feat, w1, b1, w2, b2, rb):
    n = feat.shape[0]
    grid = n // rb
    return pl.pallas_call(
        _enc_body,
        grid=(grid,),
        in_specs=[
            pl.BlockSpec((rb, D_IN), lambda i: (i, 0)),
            pl.BlockSpec((D_IN, D_MID), lambda i: (0, 0)),
            pl.BlockSpec((1, D_MID), lambda i: (0, 0)),
            pl.BlockSpec((D_MID, D_HID), lambda i: (0, 0)),
            pl.BlockSpec((1, D_HID), lambda i: (0, 0)),
        ],
        out_specs=pl.BlockSpec((rb, D_HID), lambda i: (i, 0)),
        out_shape=jax.ShapeDtypeStruct((n, D_HID), jnp.float32),
    )(feat, w1, b1, w2, b2)


def _ytab_body(x_ref, w_ref, b_ref, o_ref):
    y = jnp.dot(x_ref[...], w_ref[...],
                preferred_element_type=jnp.float32) + b_ref[...]
    o_ref[0] = y[:, :N_REL * DH]
    o_ref[1] = y[:, N_REL * DH:]


def _make_ytab(x, w, b, rb):
    w_all = w.reshape(N_REL, D_HID, 2, DH).transpose(1, 2, 0, 3) \
             .reshape(D_HID, 2 * N_REL * DH)
    b_all = b.reshape(N_REL, 2, DH).transpose(1, 0, 2).reshape(1, 2 * N_REL * DH)
    out = pl.pallas_call(
        _ytab_body,
        grid=(N // rb,),
        in_specs=[
            pl.BlockSpec((rb, D_HID), lambda i: (i, 0)),
            pl.BlockSpec((D_HID, 2 * N_REL * DH), lambda i: (0, 0)),
            pl.BlockSpec((1, 2 * N_REL * DH), lambda i: (0, 0)),
        ],
        out_specs=pl.BlockSpec((2, rb, N_REL * DH), lambda i: (0, i, 0)),
        out_shape=jax.ShapeDtypeStruct((2, N, N_REL * DH), jnp.float32),
    )(x, w_all, b_all)
    return out.reshape(2 * N_REL * N, DH)


def _eprep_body(src_ref, dst_ref, et_ref, g_ref, d_ref):
    i = pl.program_id(0)
    eid = ((i * 8 + lax.broadcasted_iota(jnp.int32, (8, CHUNK), 0)) * CHUNK
           + lax.broadcasted_iota(jnp.int32, (8, CHUNK), 1))
    valid = eid < E
    g_ref[...] = jnp.where(valid, src_ref[...] * N_REL + et_ref[...], 0)
    d_ref[...] = jnp.where(valid, dst_ref[...], N)


def _edge_prep(edge_index, edge_type):
    nbi = -(-E // (CHUNK * 8))
    ein = nbi * 8 * CHUNK

    def pad2(a):
        return jnp.concatenate([a, jnp.zeros((ein - E,), jnp.int32)]) \
                  .reshape(nbi * 8, CHUNK)
    src2, dst2, et2 = pad2(edge_index[0]), pad2(edge_index[1]), pad2(edge_type)
    clamp = lambda i: (jnp.minimum(i, nbi - 1), 0)
    return pl.pallas_call(
        _eprep_body,
        grid=(NCB // 8,),
        in_specs=[pl.BlockSpec((8, CHUNK), clamp)] * 3,
        out_specs=[pl.BlockSpec((8, CHUNK), lambda i: (i, 0))] * 2,
        out_shape=[jax.ShapeDtypeStruct((NCB, CHUNK), jnp.int32)] * 2,
    )(src2, dst2, et2)


def _combine_body(a_ref, deg_ref, x_ref, g_ref, b_ref, o_ref):
    agg = jnp.concatenate([a_ref[0], a_ref[1]], axis=1)
    agg = agg / jnp.maximum(deg_ref[...], 1.0)
    h = jax.nn.gelu(agg) + x_ref[...]
    mu = jnp.mean(h, axis=-1, keepdims=True)
    var = jnp.mean((h - mu) ** 2, axis=-1, keepdims=True)
    o_ref[...] = (h - mu) * lax.rsqrt(var + 1e-5) * g_ref[...] + b_ref[...]


def _combine(a, deg, x, ln_g, ln_b, rb):
    grid = (N // rb,)
    return pl.pallas_call(
        _combine_body,
        grid=grid,
        in_specs=[
            pl.BlockSpec((2, rb, DH), lambda i: (0, i, 0)),
            pl.BlockSpec((rb, 1), lambda i: (i, 0)),
            pl.BlockSpec((rb, D_HID), lambda i: (i, 0)),
            pl.BlockSpec((1, D_HID), lambda i: (0, 0)),
            pl.BlockSpec((1, D_HID), lambda i: (0, 0)),
        ],
        out_specs=pl.BlockSpec((rb, D_HID), lambda i: (i, 0)),
        out_shape=jax.ShapeDtypeStruct((N, D_HID), jnp.float32),
    )(a, deg, x, ln_g.reshape(1, D_HID), ln_b.reshape(1, D_HID))


def _sc_body_common(with_deg, g_hbm, dst_hbm, ytab_hbm, *rest):
    if with_deg:
        (a_out, deg_out, gwin, g2win, dwin, rows, ones, zrow, zdeg,
         acc, degacc, isem, gsem) = rest
    else:
        (a_out, gwin, g2win, dwin, rows, zrow, acc, isem, gsem) = rest
        deg_out = ones = zdeg = degacc = None

    c = lax.axis_index("c")
    s = lax.axis_index("s")

    def zrow_body(i, _):
        zrow[i, pl.ds(0, LANES)] = jnp.zeros((LANES,), jnp.float32)
        zrow[i, pl.ds(LANES, LANES)] = jnp.zeros((LANES,), jnp.float32)
        return 0
    lax.fori_loop(0, ZROWS, zrow_body, 0)
    if with_deg:
        def zdeg_body(i, _):
            zdeg[pl.ds(i * LANES, LANES)] = jnp.zeros((LANES,), jnp.float32)
            return 0
        lax.fori_loop(0, ZROWS // LANES, zdeg_body, 0)

        def ones_body(i, _):
            ones[pl.ds(i * LANES, LANES)] = jnp.ones((LANES,), jnp.float32)
            return 0
        lax.fori_loop(0, CHUNK // LANES, ones_body, 0)

    row0 = s * ROWS_PER_TILE

    def zacc_body(j, _):
        off = row0 + j * ZROWS
        pltpu.sync_copy(zrow, acc.at[pl.ds(off, ZROWS)])
        if with_deg:
            pltpu.sync_copy(zdeg, degacc.at[pl.ds(off, ZROWS)])
        return 0
    lax.fori_loop(0, ROWS_PER_TILE // ZROWS, zacc_body, 0)

    plsc.subcore_barrier()

    base_cr = s * NCHUNK
    coff = c * (N_REL * N)

    def fire_idx(j):
        sl = lax.rem(j, 3) * KW
        roff = base_cr + j * KW
        pltpu.async_copy(g_hbm.at[pl.ds(roff, KW)],
                         gwin.at[pl.ds(sl, KW)], isem.at[lax.rem(j, 3)])
        pltpu.async_copy(dst_hbm.at[pl.ds(roff, KW)],
                         dwin.at[pl.ds(sl, KW)], isem.at[lax.rem(j, 3)])

    def wait_idx(j):
        sl = lax.rem(j, 3) * KW
        roff = base_cr + j * KW
        pltpu.make_async_copy(g_hbm.at[pl.ds(roff, KW)],
                              gwin.at[pl.ds(sl, KW)],
                              isem.at[lax.rem(j, 3)]).wait()
        pltpu.make_async_copy(dst_hbm.at[pl.ds(roff, KW)],
                              dwin.at[pl.ds(sl, KW)],
                              isem.at[lax.rem(j, 3)]).wait()

    def compute_g2(j):
        bi = lax.rem(j, 3) * KW
        bg = lax.rem(j, 2) * KW

        def row_body(r, _):
            def lane_body(i, _):
                g2win[bg + r, pl.ds(i * LANES, LANES)] = (
                    gwin[bi + r, pl.ds(i * LANES, LANES)] + coff)
                return 0
            lax.fori_loop(0, CHUNK // LANES, lane_body, 0)
            return 0
        lax.fori_loop(0, KW, row_body, 0)

    def fire_gathers(j):
        bg = lax.rem(j, 2) * KW
        rj = lax.rem(j, 2)
        for k in range(KW):
            pltpu.async_copy(ytab_hbm.at[g2win.at[bg + k]],
                             rows.at[rj, k], gsem.at[rj])

    fire_idx(0)
    fire_idx(1)
    wait_idx(0)
    compute_g2(0)
    fire_gathers(0)

    def win_body(w, _):
        bw = lax.rem(w, 2)
        bi = lax.rem(w, 3) * KW

        @pl.when(w + 2 < NW)
        def _():
            fire_idx(w + 2)

        @pl.when(w + 1 < NW)
        def _():
            wait_idx(w + 1)
            compute_g2(w + 1)
            fire_gathers(w + 1)

        bg = bw * KW
        for k in range(KW):
            pltpu.make_async_copy(ytab_hbm.at[g2win.at[bg + k]],
                                  rows.at[bw, k], gsem.at[bw]).wait()
            pltpu.sync_copy(rows.at[bw, k], acc.at[dwin.at[bi + k]], add=True)
            if with_deg:
                @pl.when(c == 0)
                def _():
                    pltpu.sync_copy(ones, degacc.at[dwin.at[bi + k]],
                                    add=True)
        return 0
    lax.fori_loop(0, NW, win_body, 0)

    plsc.subcore_barrier()

    pltpu.sync_copy(acc.at[pl.ds(row0, ROWS_PER_TILE)],
                    a_out.at[c, pl.ds(row0, ROWS_PER_TILE)])
    if with_deg:
        @pl.when(c == 0)
        def _():
            pltpu.sync_copy(degacc.at[pl.ds(row0, ROWS_PER_TILE)],
                            deg_out.at[pl.ds(row0, ROWS_PER_TILE)])


def _make_sc_kernel(with_deg):
    mesh = plsc.VectorSubcoreMesh(core_axis_name="c", subcore_axis_name="s",
                                  num_cores=NUM_CORES, num_subcores=NUM_TILES)
    out_type = [jax.ShapeDtypeStruct((NUM_CORES, NP, DH), jnp.float32)]
    scratch = [
        pltpu.VMEM((3 * KW, CHUNK), jnp.int32),
        pltpu.VMEM((2 * KW, CHUNK), jnp.int32),
        pltpu.VMEM((3 * KW, CHUNK), jnp.int32),
        pltpu.VMEM((2, KW, CHUNK, DH), jnp.float32),
    ]
    if with_deg:
        out_type.append(jax.ShapeDtypeStruct((NP,), jnp.float32))
        scratch.append(pltpu.VMEM((CHUNK,), jnp.float32))
    scratch.append(pltpu.VMEM((ZROWS, DH), jnp.float32))
    if with_deg:
        scratch.append(pltpu.VMEM((ZROWS,), jnp.float32))
    scratch.append(pltpu.VMEM_SHARED((NP, DH), jnp.float32))
    if with_deg:
        scratch.append(pltpu.VMEM_SHARED((NP,), jnp.float32))
    scratch.append(pltpu.SemaphoreType.DMA((3,)))
    scratch.append(pltpu.SemaphoreType.DMA((2,)))

    def body(g_hbm, dst_hbm, ytab_hbm, *rest):
        _sc_body_common(with_deg, g_hbm, dst_hbm, ytab_hbm, *rest)

    return pl.kernel(body, out_type=tuple(out_type), mesh=mesh,
                     scratch_types=tuple(scratch),
                     compiler_params=pltpu.CompilerParams(
                         use_tc_tiling_on_sc=False))


@functools.lru_cache(maxsize=None)
def _sc_kernel(with_deg):
    return _make_sc_kernel(with_deg)


def kernel(feat_c, feat_g, feat_p, node_type, edge_index, edge_type,
           emb_W, emb_b, adapt_W, adapt_b, conv_W, conv_b, ln_g, ln_b):
    assert feat_c.shape == (N_C, D_IN) and feat_p.shape == (N_P, D_IN)
    assert edge_index.shape == (2, E)
    n_layers = conv_W.shape[0]

    parts = []
    for t, feat in enumerate((feat_c, feat_g, feat_p)):
        parts.append(_encode_type(
            feat, emb_W[t], emb_b[t].reshape(1, D_MID),
            adapt_W[t], adapt_b[t].reshape(1, D_HID), rb=1000))
    x = jnp.concatenate(parts, axis=0)

    g_pad, dst_pad = _edge_prep(edge_index, edge_type)

    deg = None
    for l in range(n_layers):
        ytab_flat = _make_ytab(x, conv_W[l], conv_b[l], rb=1000)
        if l == 0:
            a, deg = _sc_kernel(True)(g_pad, dst_pad, ytab_flat)
        else:
            (a,) = _sc_kernel(False)(g_pad, dst_pad, ytab_flat)
        x = _combine(a, deg.reshape(NP, 1), x, ln_g[l], ln_b[l], rb=1000)
    return x


# speedup vs baseline: 1.4119x; 1.4119x over previous
import functools

import jax
import jax.numpy as jnp
from jax import lax
from jax.experimental import pallas as pl
from jax.experimental.pallas import tpu as pltpu
from jax.experimental.pallas import tpu_sc as plsc

N_C, N_G, N_P = 20000, 20000, 10000
N = N_C + N_G + N_P
E = 800000
D_IN, D_MID, D_HID = 256, 256, 64
N_REL = 4
DH = D_HID // 2

NUM_CORES = 2
NUM_TILES = 16
LANES = 16

CHUNK = 128
NP = 51200
ROWS_PER_TILE = NP // NUM_TILES
E_PAD = 802816
EPT = E_PAD // NUM_TILES
NCHUNK = EPT // CHUNK
KW = 2
NW = NCHUNK // KW
NCB = E_PAD // CHUNK
ZROWS = 64


def _enc_body(f_ref, w1_ref, b1_ref, w2_ref, b2_ref, o_ref):
    h = jnp.maximum(
        jnp.dot(f_ref[...], w1_ref[...],
                preferred_element_type=jnp.float32) + b1_ref[...], 0.0)
    o_ref[...] = jnp.tanh(
        jnp.dot(h, w2_ref[...],
                preferred_element_type=jnp.float32) + b2_ref[...])


def _encode_type(feat, w1, b1, w2, b2, rb):
    n = feat.shape[0]
    grid = n // rb
    return pl.pallas_call(
        _enc_body,
        grid=(grid,),
        in_specs=[
            pl.BlockSpec((rb, D_IN), lambda i: (i, 0)),
            pl.BlockSpec((D_IN, D_MID), lambda i: (0, 0)),
            pl.BlockSpec((1, D_MID), lambda i: (0, 0)),
            pl.BlockSpec((D_MID, D_HID), lambda i: (0, 0)),
            pl.BlockSpec((1, D_HID), lambda i: (0, 0)),
        ],
        out_specs=pl.BlockSpec((rb, D_HID), lambda i: (i, 0)),
        out_shape=jax.ShapeDtypeStruct((n, D_HID), jnp.float32),
    )(feat, w1, b1, w2, b2)


def _ytab_body(x_ref, w_ref, b_ref, o_ref):
    y = jnp.dot(x_ref[...], w_ref[...],
                preferred_element_type=jnp.float32) + b_ref[...]
    o_ref[0] = y[:, :N_REL * DH]
    o_ref[1] = y[:, N_REL * DH:]


def _make_ytab(x, w, b, rb):
    w_all = w.reshape(N_REL, D_HID, 2, DH).transpose(1, 2, 0, 3) \
             .reshape(D_HID, 2 * N_REL * DH)
    b_all = b.reshape(N_REL, 2, DH).transpose(1, 0, 2).reshape(1, 2 * N_REL * DH)
    out = pl.pallas_call(
        _ytab_body,
        grid=(N // rb,),
        in_specs=[
            pl.BlockSpec((rb, D_HID), lambda i: (i, 0)),
            pl.BlockSpec((D_HID, 2 * N_REL * DH), lambda i: (0, 0)),
            pl.BlockSpec((1, 2 * N_REL * DH), lambda i: (0, 0)),
        ],
        out_specs=pl.BlockSpec((2, rb, N_REL * DH), lambda i: (0, i, 0)),
        out_shape=jax.ShapeDtypeStruct((2, N, N_REL * DH), jnp.float32),
    )(x, w_all, b_all)
    return out.reshape(2 * N_REL * N, DH)


def _combine_body(a_ref, deg_ref, x_ref, g_ref, b_ref, o_ref):
    agg = jnp.concatenate([a_ref[0], a_ref[1]], axis=1)
    agg = agg / jnp.maximum(deg_ref[...], 1.0)
    h = jax.nn.gelu(agg) + x_ref[...]
    mu = jnp.mean(h, axis=-1, keepdims=True)
    var = jnp.mean((h - mu) ** 2, axis=-1, keepdims=True)
    o_ref[...] = (h - mu) * lax.rsqrt(var + 1e-5) * g_ref[...] + b_ref[...]


def _combine(a, deg, x, ln_g, ln_b, rb):
    grid = (N // rb,)
    return pl.pallas_call(
        _combine_body,
        grid=grid,
        in_specs=[
            pl.BlockSpec((2, rb, DH), lambda i: (0, i, 0)),
            pl.BlockSpec((rb, 1), lambda i: (i, 0)),
            pl.BlockSpec((rb, D_HID), lambda i: (i, 0)),
            pl.BlockSpec((1, D_HID), lambda i: (0, 0)),
            pl.BlockSpec((1, D_HID), lambda i: (0, 0)),
        ],
        out_specs=pl.BlockSpec((rb, D_HID), lambda i: (i, 0)),
        out_shape=jax.ShapeDtypeStruct((N, D_HID), jnp.float32),
    )(a, deg, x, ln_g.reshape(1, D_HID), ln_b.reshape(1, D_HID))


def _sc_body_common(with_deg, g_hbm, dst_hbm, ytab_hbm, *rest):
    if with_deg:
        (a_out, deg_out, gwin, g2win, dwin, rows, ones, zrow, zdeg,
         acc, degacc, isem, gsem) = rest
    else:
        (a_out, gwin, g2win, dwin, rows, zrow, acc, isem, gsem) = rest
        deg_out = ones = zdeg = degacc = None

    c = lax.axis_index("c")
    s = lax.axis_index("s")

    def zrow_body(i, _):
        zrow[i, pl.ds(0, LANES)] = jnp.zeros((LANES,), jnp.float32)
        zrow[i, pl.ds(LANES, LANES)] = jnp.zeros((LANES,), jnp.float32)
        return 0
    lax.fori_loop(0, ZROWS, zrow_body, 0)
    if with_deg:
        def zdeg_body(i, _):
            zdeg[pl.ds(i * LANES, LANES)] = jnp.zeros((LANES,), jnp.float32)
            return 0
        lax.fori_loop(0, ZROWS // LANES, zdeg_body, 0)

        def ones_body(i, _):
            ones[pl.ds(i * LANES, LANES)] = jnp.ones((LANES,), jnp.float32)
            return 0
        lax.fori_loop(0, CHUNK // LANES, ones_body, 0)

    row0 = s * ROWS_PER_TILE

    def zacc_body(j, _):
        off = row0 + j * ZROWS
        pltpu.sync_copy(zrow, acc.at[pl.ds(off, ZROWS)])
        if with_deg:
            pltpu.sync_copy(zdeg, degacc.at[pl.ds(off, ZROWS)])
        return 0
    lax.fori_loop(0, ROWS_PER_TILE // ZROWS, zacc_body, 0)

    plsc.subcore_barrier()

    base_cr = s * NCHUNK
    coff = c * (N_REL * N)

    def fire_idx(j):
        sl = lax.rem(j, 3) * KW
        roff = base_cr + j * KW
        pltpu.async_copy(g_hbm.at[pl.ds(roff, KW)],
                         gwin.at[pl.ds(sl, KW)], isem.at[lax.rem(j, 3)])
        pltpu.async_copy(dst_hbm.at[pl.ds(roff, KW)],
                         dwin.at[pl.ds(sl, KW)], isem.at[lax.rem(j, 3)])

    def wait_idx(j):
        sl = lax.rem(j, 3) * KW
        roff = base_cr + j * KW
        pltpu.make_async_copy(g_hbm.at[pl.ds(roff, KW)],
                              gwin.at[pl.ds(sl, KW)],
                              isem.at[lax.rem(j, 3)]).wait()
        pltpu.make_async_copy(dst_hbm.at[pl.ds(roff, KW)],
                              dwin.at[pl.ds(sl, KW)],
                              isem.at[lax.rem(j, 3)]).wait()

    def compute_g2(j):
        bi = lax.rem(j, 3) * KW
        bg = lax.rem(j, 2) * KW

        def row_body(r, _):
            def lane_body(i, _):
                g2win[bg + r, pl.ds(i * LANES, LANES)] = (
                    gwin[bi + r, pl.ds(i * LANES, LANES)] + coff)
                return 0
            lax.fori_loop(0, CHUNK // LANES, lane_body, 0)
            return 0
        lax.fori_loop(0, KW, row_body, 0)

    def fire_gathers(j):
        bg = lax.rem(j, 2) * KW
        rj = lax.rem(j, 2)
        for k in range(KW):
            pltpu.async_copy(ytab_hbm.at[g2win.at[bg + k]],
                             rows.at[rj, k], gsem.at[rj])

    fire_idx(0)
    fire_idx(1)
    wait_idx(0)
    compute_g2(0)
    fire_gathers(0)

    def win_body(w, _):
        bw = lax.rem(w, 2)
        bi = lax.rem(w, 3) * KW

        @pl.when(w + 2 < NW)
        def _():
            fire_idx(w + 2)

        @pl.when(w + 1 < NW)
        def _():
            wait_idx(w + 1)
            compute_g2(w + 1)
            fire_gathers(w + 1)

        bg = bw * KW
        for k in range(KW):
            pltpu.make_async_copy(ytab_hbm.at[g2win.at[bg + k]],
                                  rows.at[bw, k], gsem.at[bw]).wait()
            pltpu.sync_copy(rows.at[bw, k], acc.at[dwin.at[bi + k]], add=True)
            if with_deg:
                @pl.when(c == 0)
                def _():
                    pltpu.sync_copy(ones, degacc.at[dwin.at[bi + k]],
                                    add=True)
        return 0
    lax.fori_loop(0, NW, win_body, 0)

    plsc.subcore_barrier()

    pltpu.sync_copy(acc.at[pl.ds(row0, ROWS_PER_TILE)],
                    a_out.at[c, pl.ds(row0, ROWS_PER_TILE)])
    if with_deg:
        @pl.when(c == 0)
        def _():
            pltpu.sync_copy(degacc.at[pl.ds(row0, ROWS_PER_TILE)],
                            deg_out.at[pl.ds(row0, ROWS_PER_TILE)])


def _make_sc_kernel(with_deg):
    mesh = plsc.VectorSubcoreMesh(core_axis_name="c", subcore_axis_name="s",
                                  num_cores=NUM_CORES, num_subcores=NUM_TILES)
    out_type = [jax.ShapeDtypeStruct((NUM_CORES, NP, DH), jnp.float32)]
    scratch = [
        pltpu.VMEM((3 * KW, CHUNK), jnp.int32),
        pltpu.VMEM((2 * KW, CHUNK), jnp.int32),
        pltpu.VMEM((3 * KW, CHUNK), jnp.int32),
        pltpu.VMEM((2, KW, CHUNK, DH), jnp.float32),
    ]
    if with_deg:
        out_type.append(jax.ShapeDtypeStruct((NP,), jnp.float32))
        scratch.append(pltpu.VMEM((CHUNK,), jnp.float32))
    scratch.append(pltpu.VMEM((ZROWS, DH), jnp.float32))
    if with_deg:
        scratch.append(pltpu.VMEM((ZROWS,), jnp.float32))
    scratch.append(pltpu.VMEM_SHARED((NP, DH), jnp.float32))
    if with_deg:
        scratch.append(pltpu.VMEM_SHARED((NP,), jnp.float32))
    scratch.append(pltpu.SemaphoreType.DMA((3,)))
    scratch.append(pltpu.SemaphoreType.DMA((2,)))

    def body(g_hbm, dst_hbm, ytab_hbm, *rest):
        _sc_body_common(with_deg, g_hbm, dst_hbm, ytab_hbm, *rest)

    return pl.kernel(body, out_type=tuple(out_type), mesh=mesh,
                     scratch_types=tuple(scratch),
                     compiler_params=pltpu.CompilerParams(
                         use_tc_tiling_on_sc=False))


@functools.lru_cache(maxsize=None)
def _sc_kernel(with_deg):
    return _make_sc_kernel(with_deg)


def kernel(feat_c, feat_g, feat_p, node_type, edge_index, edge_type,
           emb_W, emb_b, adapt_W, adapt_b, conv_W, conv_b, ln_g, ln_b):
    assert feat_c.shape == (N_C, D_IN) and feat_p.shape == (N_P, D_IN)
    assert edge_index.shape == (2, E)
    n_layers = conv_W.shape[0]

    parts = []
    for t, feat in enumerate((feat_c, feat_g, feat_p)):
        parts.append(_encode_type(
            feat, emb_W[t], emb_b[t].reshape(1, D_MID),
            adapt_W[t], adapt_b[t].reshape(1, D_HID), rb=1000))
    x = jnp.concatenate(parts, axis=0)

    src = edge_index[0]
    dst = edge_index[1]
    g = src * N_REL + edge_type
    pad = E_PAD - E
    g_pad = jnp.concatenate([g, jnp.zeros((pad,), jnp.int32)]) \
               .reshape(NCB, CHUNK)
    dst_pad = jnp.concatenate([dst, jnp.full((pad,), N, jnp.int32)]) \
                 .reshape(NCB, CHUNK)

    deg = None
    for l in range(n_layers):
        ytab_flat = _make_ytab(x, conv_W[l], conv_b[l], rb=1000)
        if l == 0:
            a, deg = _sc_kernel(True)(g_pad, dst_pad, ytab_flat)
        else:
            (a,) = _sc_kernel(False)(g_pad, dst_pad, ytab_flat)
        x = _combine(a, deg.reshape(NP, 1), x, ln_g[l], ln_b[l], rb=1000)
    return x
